# Initial kernel scaffold; baseline (speedup 1.0000x reference)
import jax
import jax.numpy as jnp
from jax.experimental import pallas as pl


def kernel(x, edge_index, edge_attr, batch, global_features, params):
    raise NotImplementedError("write your pallas kernel here")


# speedup vs baseline: 3.0407x; 3.0407x over previous
import functools

import jax
import jax.numpy as jnp
from jax import lax
from jax.experimental import pallas as pl
from jax.experimental.pallas import tpu as pltpu
from jax.experimental.pallas import tpu_sc as plsc

N = 10000
NPAD = 10240
E = 320000
ECHUNK = 128
IDX_BLK = 16
NBLK = 5
EPW = IDX_BLK * NBLK
EPAD = 32 * EPW * ECHUNK
NROWS_PER_TILE = NPAD // 16


def _edge_agg(h, src2d, dst2d):
    mesh = plsc.VectorSubcoreMesh(core_axis_name="c", subcore_axis_name="s")

    @functools.partial(
        pl.kernel,
        out_type=[jax.ShapeDtypeStruct((NPAD, 128), jnp.float32),
                  jax.ShapeDtypeStruct((NPAD, 128), jnp.float32)],
        mesh=mesh,
        scratch_types=[
            pltpu.VMEM_SHARED((NPAD, 128), jnp.float32),
            pltpu.VMEM((IDX_BLK, ECHUNK), jnp.int32),
            pltpu.VMEM((IDX_BLK, ECHUNK), jnp.int32),
            pltpu.VMEM((2, ECHUNK, 128), jnp.float32),
            pltpu.SemaphoreType.DMA,
            pltpu.SemaphoreType.DMA,
        ],
    )
    def k(h_hbm, src_hbm, dst_hbm, out0, out1, agg, src_v, dst_v, rows, sem0, sem1):
        c = lax.axis_index("c")
        s = lax.axis_index("s")
        w = c * 16 + s
        tbase = s * NROWS_PER_TILE

        zero16 = jnp.zeros((16,), jnp.float32)

        def zrow(r, carry):
            for l in range(8):
                rows[0, r, pl.ds(l * 16, 16)] = zero16
            return carry
        lax.fori_loop(0, ECHUNK, zrow, 0)

        def zagg(kk, carry):
            pltpu.sync_copy(rows.at[0], agg.at[pl.ds(tbase + kk * 128, 128)])
            return carry
        lax.fori_loop(0, NPAD // 16 // 128, zagg, 0)

        plsc.subcore_barrier()

        ebase = w * EPW
        npairs = IDX_BLK // 2

        def blk(b, carry):
            pltpu.sync_copy(src_hbm.at[pl.ds(ebase + b * IDX_BLK, IDX_BLK)], src_v)
            pltpu.sync_copy(dst_hbm.at[pl.ds(ebase + b * IDX_BLK, IDX_BLK)], dst_v)
            pltpu.async_copy(h_hbm.at[src_v.at[0]], rows.at[0], sem0)

            def pair(p, carry2):
                j0 = 2 * p
                pltpu.async_copy(h_hbm.at[src_v.at[j0 + 1]], rows.at[1], sem1)
                pltpu.make_async_copy(
                    h_hbm.at[src_v.at[j0]], rows.at[0], sem0).wait()
                pltpu.sync_copy(rows.at[0], agg.at[dst_v.at[j0]], add=True)

                @pl.when(p < npairs - 1)
                def _():
                    pltpu.async_copy(h_hbm.at[src_v.at[j0 + 2]], rows.at[0], sem0)

                pltpu.make_async_copy(
                    h_hbm.at[src_v.at[j0 + 1]], rows.at[1], sem1).wait()
                pltpu.sync_copy(rows.at[1], agg.at[dst_v.at[j0 + 1]], add=True)
                return carry2
            lax.fori_loop(0, npairs, pair, 0)
            return carry
        lax.fori_loop(0, NBLK, blk, 0)

        plsc.subcore_barrier()

        def outcp(kk, carry):
            pltpu.sync_copy(agg.at[pl.ds(tbase + kk * 128, 128)], rows.at[0])

            @pl.when(c == 0)
            def _():
                pltpu.sync_copy(rows.at[0], out0.at[pl.ds(tbase + kk * 128, 128)])

            @pl.when(c == 1)
            def _():
                pltpu.sync_copy(rows.at[0], out1.at[pl.ds(tbase + kk * 128, 128)])
            return carry
        lax.fori_loop(0, NPAD // 16 // 128, outcp, 0)

    return k(h, src2d, dst2d)


POOL_CH = 128


def _pool(h, batch_pad):
    mesh = plsc.VectorSubcoreMesh(core_axis_name="c", subcore_axis_name="s")

    @functools.partial(
        pl.kernel,
        out_type=jax.ShapeDtypeStruct((64, 384), jnp.float32),
        mesh=mesh,
        scratch_types=[
            pltpu.VMEM((NPAD,), jnp.int32),
            pltpu.VMEM((POOL_CH, 128), jnp.float32),
            pltpu.VMEM((2, 384), jnp.float32),
        ],
    )
    def k(h_hbm, b_hbm, out, bat_v, hrows, pool_v):
        c = lax.axis_index("c")
        s = lax.axis_index("s")
        w = c * 16 + s
        g0 = 2 * w

        pltpu.sync_copy(b_hbm, bat_v)

        def lower_bound(g):
            def bb(_, lohi):
                lo, hi = lohi
                mid = (lo + hi) // 2
                v = bat_v[pl.ds(mid * 16, 16)]
                big = v[0] >= g
                return (jnp.where(big, lo, mid + 1), jnp.where(big, mid, hi))
            bstar, _ = lax.fori_loop(
                0, 10, bb, (jnp.int32(0), jnp.int32(NPAD // 16)))
            bprev = jnp.maximum(bstar - 1, 0)
            v = bat_v[pl.ds(bprev * 16, 16)]
            cnt = jnp.int32(0)
            for l in range(16):
                cnt = cnt + jnp.where(v[l] < g, 1, 0).astype(jnp.int32)
            return bprev * 16 + cnt

        bounds = (lower_bound(g0), lower_bound(g0 + 1), lower_bound(g0 + 2))

        for gi in range(2):
            start = bounds[gi]
            end = bounds[gi + 1]
            cnt = end - start
            abase = (start // 8) * 8
            nch = (end - abase + (POOL_CH - 1)) // POOL_CH

            sums0 = tuple(jnp.zeros((16,), jnp.float32) for _ in range(8))
            maxs0 = tuple(jnp.zeros((16,), jnp.float32) for _ in range(8))

            def chunk(ci, accs):
                base = abase + ci * POOL_CH
                pltpu.sync_copy(h_hbm.at[pl.ds(base, POOL_CH)], hrows)
                lo = jnp.maximum(start - base, 0)
                hi = jnp.minimum(POOL_CH, end - base)

                def row(r, accs2):
                    sums, maxs = accs2
                    ns, nm = [], []
                    for l in range(8):
                        v = hrows[r, pl.ds(l * 16, 16)]
                        ns.append(sums[l] + v)
                        nm.append(jnp.maximum(maxs[l], v))
                    return (tuple(ns), tuple(nm))
                return lax.fori_loop(lo, hi, row, accs)
            sums, maxs = lax.fori_loop(0, nch, chunk, (sums0, maxs0))

            cnt_f = jnp.broadcast_to(cnt, (16,)).astype(jnp.float32)
            inv = 1.0 / jnp.maximum(cnt_f, 1.0)
            for l in range(8):
                pool_v[gi, pl.ds(l * 16, 16)] = sums[l] * inv
                pool_v[gi, pl.ds(128 + l * 16, 16)] = maxs[l]
                pool_v[gi, pl.ds(256 + l * 16, 16)] = sums[l]

        pltpu.sync_copy(pool_v, out.at[pl.ds(g0, 2)])

    return k(h, batch_pad)


_BM = 1024


def _encoder(x, W1, b1, W2, b2):
    def body(x_ref, w1_ref, b1_ref, w2_ref, b2_ref, o_ref):
        h = jnp.maximum(
            jnp.dot(x_ref[...], w1_ref[...],
                    preferred_element_type=jnp.float32) + b1_ref[...], 0.0)
        o_ref[...] = jnp.maximum(
            jnp.dot(h, w2_ref[...],
                    preferred_element_type=jnp.float32) + b2_ref[...], 0.0)

    return pl.pallas_call(
        body,
        grid=(NPAD // _BM,),
        in_specs=[
            pl.BlockSpec((_BM, 128), lambda i: (i, 0)),
            pl.BlockSpec((128, 128), lambda i: (0, 0)),
            pl.BlockSpec((1, 128), lambda i: (0, 0)),
            pl.BlockSpec((128, 128), lambda i: (0, 0)),
            pl.BlockSpec((1, 128), lambda i: (0, 0)),
        ],
        out_specs=pl.BlockSpec((_BM, 128), lambda i: (i, 0)),
        out_shape=jax.ShapeDtypeStruct((NPAD, 128), jnp.float32),
    )(x, W1, b1, W2, b2)


def _combine(p0, p1, h, Wrel, brel, Wroot):
    def body(p0_ref, p1_ref, h_ref, wr_ref, br_ref, wo_ref, o_ref):
        agg = p0_ref[...] + p1_ref[...]
        hv = h_ref[...]
        out = (jnp.dot(agg, wr_ref[...], preferred_element_type=jnp.float32)
               + br_ref[...]
               + jnp.dot(hv, wo_ref[...], preferred_element_type=jnp.float32))
        o_ref[...] = jnp.maximum(out, 0.0) + hv

    return pl.pallas_call(
        body,
        grid=(NPAD // _BM,),
        in_specs=[
            pl.BlockSpec((_BM, 128), lambda i: (i, 0)),
            pl.BlockSpec((_BM, 128), lambda i: (i, 0)),
            pl.BlockSpec((_BM, 128), lambda i: (i, 0)),
            pl.BlockSpec((128, 128), lambda i: (0, 0)),
            pl.BlockSpec((1, 128), lambda i: (0, 0)),
            pl.BlockSpec((128, 128), lambda i: (0, 0)),
        ],
        out_specs=pl.BlockSpec((_BM, 128), lambda i: (i, 0)),
        out_shape=jax.ShapeDtypeStruct((NPAD, 128), jnp.float32),
    )(p0, p1, h, Wrel, brel, Wroot)


def _head(pooled, gf, wg1, bg1, wg2, bg2, wp, wg, b1, w2, b2, w3, b3, w4, b4):
    def body(pool_ref, gf_ref, wg1_ref, bg1_ref, wg2_ref, bg2_ref, wp_ref,
             wgc_ref, b1_ref, w2_ref, b2_ref, w3_ref, b3_ref, w4_ref, b4_ref,
             o_ref):
        dot = functools.partial(jnp.dot, preferred_element_type=jnp.float32)
        g = jnp.maximum(dot(gf_ref[...], wg1_ref[...]) + bg1_ref[...], 0.0)
        g = jnp.maximum(dot(g, wg2_ref[...]) + bg2_ref[...], 0.0)
        z = jnp.maximum(
            dot(pool_ref[...], wp_ref[...]) + dot(g, wgc_ref[...])
            + b1_ref[...], 0.0)
        z = jnp.maximum(dot(z, w2_ref[...]) + b2_ref[...], 0.0)
        z = jnp.maximum(dot(z, w3_ref[...]) + b3_ref[...], 0.0)
        o_ref[...] = dot(z, w4_ref[...]) + b4_ref[...]

    return pl.pallas_call(
        body,
        out_shape=jax.ShapeDtypeStruct((64, 128), jnp.float32),
    )(pooled, gf, wg1, bg1, wg2, bg2, wp, wg, b1, w2, b2, w3, b3, w4, b4)


def kernel(x, edge_index, edge_attr, batch, global_features, params):
    del edge_attr

    xp = jnp.pad(x, ((0, NPAD - N), (0, 0)))
    src = jnp.concatenate(
        [edge_index[0], jnp.zeros((EPAD - E,), jnp.int32)]).reshape(-1, ECHUNK)
    dst = jnp.concatenate(
        [edge_index[1], jnp.full((EPAD - E,), N, jnp.int32)]).reshape(-1, ECHUNK)
    batch_pad = jnp.concatenate(
        [batch, jnp.full((NPAD - N,), 64, jnp.int32)])

    row = lambda b: b.reshape(1, -1)

    W1, b1 = params['ne1']
    W2, b2 = params['ne2']
    h = _encoder(xp, W1, row(b1), W2, row(b2))

    for layer in params['convs']:
        Wrel, brel = layer['rel']
        p0, p1 = _edge_agg(h, src, dst)
        h = _combine(p0, p1, h, Wrel, row(brel), layer['root'])

    pooled = _pool(h, batch_pad)

    wg1, bg1 = params['ge1']
    wg2, bg2 = params['ge2']
    (w1c, b1c), (w2c, b2c), (w3c, b3c), (w4c, b4c) = params['cls']
    wp, wg = w1c[:384], w1c[384:]
    w4p = jnp.pad(w4c, ((0, 0), (0, 128 - w4c.shape[1])))
    b4p = jnp.pad(b4c, ((0, 128 - b4c.shape[0]),))

    out = _head(pooled, global_features, wg1, row(bg1), wg2, row(bg2),
                wp, wg, row(b1c), w2c, row(b2c), w3c, row(b3c), w4p, row(b4p))
    return out[:, :2]


# speedup vs baseline: 10.5712x; 3.4766x over previous
import functools

import jax
import jax.numpy as jnp
from jax import lax
from jax.experimental import pallas as pl
from jax.experimental.pallas import tpu as pltpu
from jax.experimental.pallas import tpu_sc as plsc

N = 10000
NPAD = 10240
E = 320000
ECHUNK = 128
IDX_BLK = 16
NBLK = 5
EPW = IDX_BLK * NBLK
EPAD = 32 * EPW * ECHUNK
NROWS_PER_TILE = NPAD // 16


def _edge_agg(h, src2d, dst2d):
    mesh = plsc.VectorSubcoreMesh(core_axis_name="c", subcore_axis_name="s")

    @functools.partial(
        pl.kernel,
        out_type=[jax.ShapeDtypeStruct((NPAD, 128), jnp.float32),
                  jax.ShapeDtypeStruct((NPAD, 128), jnp.float32)],
        mesh=mesh,
        scratch_types=[
            pltpu.VMEM_SHARED((NPAD, 128), jnp.float32),
            pltpu.VMEM((IDX_BLK, ECHUNK), jnp.int32),
            pltpu.VMEM((IDX_BLK, ECHUNK), jnp.int32),
            pltpu.VMEM((2, ECHUNK, 128), jnp.float32),
            pltpu.SemaphoreType.DMA,
            pltpu.SemaphoreType.DMA,
        ],
    )
    def k(h_hbm, src_hbm, dst_hbm, out0, out1, agg, src_v, dst_v, rows, sem0, sem1):
        c = lax.axis_index("c")
        s = lax.axis_index("s")
        w = c * 16 + s
        tbase = s * NROWS_PER_TILE

        zero16 = jnp.zeros((16,), jnp.float32)

        def zrow(r, carry):
            for l in range(8):
                rows[0, r, pl.ds(l * 16, 16)] = zero16
            return carry
        lax.fori_loop(0, ECHUNK, zrow, 0)

        def zagg(kk, carry):
            pltpu.sync_copy(rows.at[0], agg.at[pl.ds(tbase + kk * 128, 128)])
            return carry
        lax.fori_loop(0, NPAD // 16 // 128, zagg, 0)

        plsc.subcore_barrier()

        ebase = w * EPW
        npairs = IDX_BLK // 2

        def blk(b, carry):
            pltpu.sync_copy(src_hbm.at[pl.ds(ebase + b * IDX_BLK, IDX_BLK)], src_v)
            pltpu.sync_copy(dst_hbm.at[pl.ds(ebase + b * IDX_BLK, IDX_BLK)], dst_v)
            pltpu.async_copy(h_hbm.at[src_v.at[0]], rows.at[0], sem0)

            def pair(p, carry2):
                j0 = 2 * p
                pltpu.async_copy(h_hbm.at[src_v.at[j0 + 1]], rows.at[1], sem1)
                pltpu.make_async_copy(
                    h_hbm.at[src_v.at[j0]], rows.at[0], sem0).wait()
                pltpu.sync_copy(rows.at[0], agg.at[dst_v.at[j0]], add=True)

                @pl.when(p < npairs - 1)
                def _():
                    pltpu.async_copy(h_hbm.at[src_v.at[j0 + 2]], rows.at[0], sem0)

                pltpu.make_async_copy(
                    h_hbm.at[src_v.at[j0 + 1]], rows.at[1], sem1).wait()
                pltpu.sync_copy(rows.at[1], agg.at[dst_v.at[j0 + 1]], add=True)
                return carry2
            lax.fori_loop(0, npairs, pair, 0)
            return carry
        lax.fori_loop(0, NBLK, blk, 0)

        plsc.subcore_barrier()

        def outcp(kk, carry):
            pltpu.sync_copy(agg.at[pl.ds(tbase + kk * 128, 128)], rows.at[0])

            @pl.when(c == 0)
            def _():
                pltpu.sync_copy(rows.at[0], out0.at[pl.ds(tbase + kk * 128, 128)])

            @pl.when(c == 1)
            def _():
                pltpu.sync_copy(rows.at[0], out1.at[pl.ds(tbase + kk * 128, 128)])
            return carry
        lax.fori_loop(0, NPAD // 16 // 128, outcp, 0)

    return k(h, src2d, dst2d)


POOL_CH = 128


def _pool(h, batch_pad):
    mesh = plsc.VectorSubcoreMesh(core_axis_name="c", subcore_axis_name="s")

    @functools.partial(
        pl.kernel,
        out_type=jax.ShapeDtypeStruct((64, 384), jnp.float32),
        mesh=mesh,
        scratch_types=[
            pltpu.VMEM((NPAD,), jnp.int32),
            pltpu.VMEM((POOL_CH, 128), jnp.float32),
            pltpu.VMEM((2, 384), jnp.float32),
        ],
    )
    def k(h_hbm, b_hbm, out, bat_v, hrows, pool_v):
        c = lax.axis_index("c")
        s = lax.axis_index("s")
        w = c * 16 + s
        g0 = 2 * w

        pltpu.sync_copy(b_hbm, bat_v)

        def lower_bound(g):
            def bb(_, lohi):
                lo, hi = lohi
                mid = (lo + hi) // 2
                v = bat_v[pl.ds(mid * 16, 16)]
                big = v[0] >= g
                return (jnp.where(big, lo, mid + 1), jnp.where(big, mid, hi))
            bstar, _ = lax.fori_loop(
                0, 10, bb, (jnp.int32(0), jnp.int32(NPAD // 16)))
            bprev = jnp.maximum(bstar - 1, 0)
            v = bat_v[pl.ds(bprev * 16, 16)]
            cnt = jnp.int32(0)
            for l in range(16):
                cnt = cnt + jnp.where(v[l] < g, 1, 0).astype(jnp.int32)
            return bprev * 16 + cnt

        bounds = (lower_bound(g0), lower_bound(g0 + 1), lower_bound(g0 + 2))

        for gi in range(2):
            start = bounds[gi]
            end = bounds[gi + 1]
            cnt = end - start
            abase = (start // 8) * 8
            nch = (end - abase + (POOL_CH - 1)) // POOL_CH

            sums0 = tuple(jnp.zeros((16,), jnp.float32) for _ in range(8))
            maxs0 = tuple(jnp.zeros((16,), jnp.float32) for _ in range(8))

            def chunk(ci, accs):
                base = abase + ci * POOL_CH
                pltpu.sync_copy(h_hbm.at[pl.ds(base, POOL_CH)], hrows)
                lo = jnp.maximum(start - base, 0)
                hi = jnp.minimum(POOL_CH, end - base)

                def row(r, accs2):
                    sums, maxs = accs2
                    ns, nm = [], []
                    for l in range(8):
                        v = hrows[r, pl.ds(l * 16, 16)]
                        ns.append(sums[l] + v)
                        nm.append(jnp.maximum(maxs[l], v))
                    return (tuple(ns), tuple(nm))
                return lax.fori_loop(lo, hi, row, accs)
            sums, maxs = lax.fori_loop(0, nch, chunk, (sums0, maxs0))

            cnt_f = jnp.broadcast_to(cnt, (16,)).astype(jnp.float32)
            inv = 1.0 / jnp.maximum(cnt_f, 1.0)
            for l in range(8):
                pool_v[gi, pl.ds(l * 16, 16)] = sums[l] * inv
                pool_v[gi, pl.ds(128 + l * 16, 16)] = maxs[l]
                pool_v[gi, pl.ds(256 + l * 16, 16)] = sums[l]

        pltpu.sync_copy(pool_v, out.at[pl.ds(g0, 2)])

    return k(h, batch_pad)


_BM = 1024


def _encoder(x, W1, b1, W2, b2):
    def body(x_ref, w1_ref, b1_ref, w2_ref, b2_ref, o_ref):
        h = jnp.maximum(
            jnp.dot(x_ref[...], w1_ref[...],
                    preferred_element_type=jnp.float32) + b1_ref[...], 0.0)
        o_ref[...] = jnp.maximum(
            jnp.dot(h, w2_ref[...],
                    preferred_element_type=jnp.float32) + b2_ref[...], 0.0)

    return pl.pallas_call(
        body,
        grid=(NPAD // _BM,),
        in_specs=[
            pl.BlockSpec((_BM, 128), lambda i: (i, 0)),
            pl.BlockSpec((128, 128), lambda i: (0, 0)),
            pl.BlockSpec((1, 128), lambda i: (0, 0)),
            pl.BlockSpec((128, 128), lambda i: (0, 0)),
            pl.BlockSpec((1, 128), lambda i: (0, 0)),
        ],
        out_specs=pl.BlockSpec((_BM, 128), lambda i: (i, 0)),
        out_shape=jax.ShapeDtypeStruct((NPAD, 128), jnp.float32),
    )(x, W1, b1, W2, b2)


def _combine(p0, p1, h, Wrel, brel, Wroot):
    def body(p0_ref, p1_ref, h_ref, wr_ref, br_ref, wo_ref, o_ref):
        agg = p0_ref[...] + p1_ref[...]
        hv = h_ref[...]
        out = (jnp.dot(agg, wr_ref[...], preferred_element_type=jnp.float32)
               + br_ref[...]
               + jnp.dot(hv, wo_ref[...], preferred_element_type=jnp.float32))
        o_ref[...] = jnp.maximum(out, 0.0) + hv

    return pl.pallas_call(
        body,
        grid=(NPAD // _BM,),
        in_specs=[
            pl.BlockSpec((_BM, 128), lambda i: (i, 0)),
            pl.BlockSpec((_BM, 128), lambda i: (i, 0)),
            pl.BlockSpec((_BM, 128), lambda i: (i, 0)),
            pl.BlockSpec((128, 128), lambda i: (0, 0)),
            pl.BlockSpec((1, 128), lambda i: (0, 0)),
            pl.BlockSpec((128, 128), lambda i: (0, 0)),
        ],
        out_specs=pl.BlockSpec((_BM, 128), lambda i: (i, 0)),
        out_shape=jax.ShapeDtypeStruct((NPAD, 128), jnp.float32),
    )(p0, p1, h, Wrel, brel, Wroot)


def _head(pooled, gf, wg1, bg1, wg2, bg2, wp, wg, b1, w2, b2, w3, b3, w4, b4):
    def body(pool_ref, gf_ref, wg1_ref, bg1_ref, wg2_ref, bg2_ref, wp_ref,
             wgc_ref, b1_ref, w2_ref, b2_ref, w3_ref, b3_ref, w4_ref, b4_ref,
             o_ref):
        dot = functools.partial(jnp.dot, preferred_element_type=jnp.float32)
        g = jnp.maximum(dot(gf_ref[...], wg1_ref[...]) + bg1_ref[...], 0.0)
        g = jnp.maximum(dot(g, wg2_ref[...]) + bg2_ref[...], 0.0)
        z = jnp.maximum(
            dot(pool_ref[...], wp_ref[...]) + dot(g, wgc_ref[...])
            + b1_ref[...], 0.0)
        z = jnp.maximum(dot(z, w2_ref[...]) + b2_ref[...], 0.0)
        z = jnp.maximum(dot(z, w3_ref[...]) + b3_ref[...], 0.0)
        o_ref[...] = dot(z, w4_ref[...]) + b4_ref[...]

    return pl.pallas_call(
        body,
        out_shape=jax.ShapeDtypeStruct((64, 128), jnp.float32),
    )(pooled, gf, wg1, bg1, wg2, bg2, wp, wg, b1, w2, b2, w3, b3, w4, b4)


def kernel(x, edge_index, edge_attr, batch, global_features, params):
    del edge_attr

    xp = jnp.pad(x, ((0, NPAD - N), (0, 0)))
    pad_i = jnp.arange(EPAD - E, dtype=jnp.int32)
    src = jnp.concatenate(
        [edge_index[0], pad_i % N]).reshape(-1, ECHUNK)
    dst = jnp.concatenate(
        [edge_index[1], N + pad_i % (NPAD - N)]).reshape(-1, ECHUNK)
    batch_pad = jnp.concatenate(
        [batch, jnp.full((NPAD - N,), 64, jnp.int32)])

    row = lambda b: b.reshape(1, -1)

    W1, b1 = params['ne1']
    W2, b2 = params['ne2']
    h = _encoder(xp, W1, row(b1), W2, row(b2))

    for layer in params['convs']:
        Wrel, brel = layer['rel']
        p0, p1 = _edge_agg(h, src, dst)
        h = _combine(p0, p1, h, Wrel, row(brel), layer['root'])

    pooled = _pool(h, batch_pad)

    wg1, bg1 = params['ge1']
    wg2, bg2 = params['ge2']
    (w1c, b1c), (w2c, b2c), (w3c, b3c), (w4c, b4c) = params['cls']
    wp, wg = w1c[:384], w1c[384:]
    w4p = jnp.pad(w4c, ((0, 0), (0, 128 - w4c.shape[1])))
    b4p = jnp.pad(b4c, ((0, 128 - b4c.shape[0]),))

    out = _head(pooled, global_features, wg1, row(bg1), wg2, row(bg2),
                wp, wg, row(b1c), w2c, row(b2c), w3c, row(b3c), w4p, row(b4p))
    return out[:, :2]


# speedup vs baseline: 11.3354x; 1.0723x over previous
import functools

import jax
import jax.numpy as jnp
from jax import lax
from jax.experimental import pallas as pl
from jax.experimental.pallas import tpu as pltpu
from jax.experimental.pallas import tpu_sc as plsc

N = 10000
NPAD = 10240
E = 320000
ECHUNK = 128
IDX_BLK = 40
NBLK = 2
EPW = IDX_BLK * NBLK
EPAD = 32 * EPW * ECHUNK
NROWS_PER_TILE = NPAD // 16


def _edge_agg(h, src2d, dst2d):
    mesh = plsc.VectorSubcoreMesh(core_axis_name="c", subcore_axis_name="s")

    @functools.partial(
        pl.kernel,
        out_type=[jax.ShapeDtypeStruct((NPAD, 128), jnp.float32),
                  jax.ShapeDtypeStruct((NPAD, 128), jnp.float32)],
        mesh=mesh,
        scratch_types=[
            pltpu.VMEM_SHARED((NPAD, 128), jnp.float32),
            pltpu.VMEM((IDX_BLK, ECHUNK), jnp.int32),
            pltpu.VMEM((IDX_BLK, ECHUNK), jnp.int32),
            pltpu.VMEM((2, ECHUNK, 128), jnp.float32),
            pltpu.SemaphoreType.DMA,
            pltpu.SemaphoreType.DMA,
        ],
    )
    def k(h_hbm, src_hbm, dst_hbm, out0, out1, agg, src_v, dst_v, rows, sem0, sem1):
        c = lax.axis_index("c")
        s = lax.axis_index("s")
        w = c * 16 + s
        tbase = s * NROWS_PER_TILE

        zero16 = jnp.zeros((16,), jnp.float32)

        def zrow(r, carry):
            for l in range(8):
                rows[0, r, pl.ds(l * 16, 16)] = zero16
            return carry
        lax.fori_loop(0, ECHUNK, zrow, 0)

        def zagg(kk, carry):
            pltpu.sync_copy(rows.at[0], agg.at[pl.ds(tbase + kk * 128, 128)])
            return carry
        lax.fori_loop(0, NPAD // 16 // 128, zagg, 0)

        plsc.subcore_barrier()

        ebase = w * EPW
        npairs = IDX_BLK // 2

        def blk(b, carry):
            pltpu.sync_copy(src_hbm.at[pl.ds(ebase + b * IDX_BLK, IDX_BLK)], src_v)
            pltpu.sync_copy(dst_hbm.at[pl.ds(ebase + b * IDX_BLK, IDX_BLK)], dst_v)
            pltpu.async_copy(h_hbm.at[src_v.at[0]], rows.at[0], sem0)

            def pair(p, carry2):
                j0 = 2 * p
                pltpu.async_copy(h_hbm.at[src_v.at[j0 + 1]], rows.at[1], sem1)
                pltpu.make_async_copy(
                    h_hbm.at[src_v.at[j0]], rows.at[0], sem0).wait()
                pltpu.sync_copy(rows.at[0], agg.at[dst_v.at[j0]], add=True)

                @pl.when(p < npairs - 1)
                def _():
                    pltpu.async_copy(h_hbm.at[src_v.at[j0 + 2]], rows.at[0], sem0)

                pltpu.make_async_copy(
                    h_hbm.at[src_v.at[j0 + 1]], rows.at[1], sem1).wait()
                pltpu.sync_copy(rows.at[1], agg.at[dst_v.at[j0 + 1]], add=True)
                return carry2
            lax.fori_loop(0, npairs, pair, 0)
            return carry
        lax.fori_loop(0, NBLK, blk, 0)

        plsc.subcore_barrier()

        nout = NPAD // 16 // 128

        def stage(kk, buf):
            pltpu.sync_copy(agg.at[pl.ds(tbase + kk * 128, 128)], rows.at[buf])

        def drain(kk, buf, sem):
            @pl.when(c == 0)
            def _():
                pltpu.async_copy(
                    rows.at[buf], out0.at[pl.ds(tbase + kk * 128, 128)], sem)

            @pl.when(c == 1)
            def _():
                pltpu.async_copy(
                    rows.at[buf], out1.at[pl.ds(tbase + kk * 128, 128)], sem)

        def wait_drain(kk, buf, sem):
            @pl.when(c == 0)
            def _():
                pltpu.make_async_copy(
                    rows.at[buf], out0.at[pl.ds(tbase + kk * 128, 128)], sem).wait()

            @pl.when(c == 1)
            def _():
                pltpu.make_async_copy(
                    rows.at[buf], out1.at[pl.ds(tbase + kk * 128, 128)], sem).wait()

        for kk in range(nout):
            buf = kk % 2
            sem = sem0 if buf == 0 else sem1
            if kk >= 2:
                wait_drain(kk - 2, buf, sem)
            stage(kk, buf)
            drain(kk, buf, sem)
        for kk in range(max(nout - 2, 0), nout):
            buf = kk % 2
            wait_drain(kk, buf, sem0 if buf == 0 else sem1)

    return k(h, src2d, dst2d)


POOL_CH = 128


def _pool(h, batch_pad):
    mesh = plsc.VectorSubcoreMesh(core_axis_name="c", subcore_axis_name="s")

    @functools.partial(
        pl.kernel,
        out_type=jax.ShapeDtypeStruct((64, 384), jnp.float32),
        mesh=mesh,
        scratch_types=[
            pltpu.VMEM((NPAD,), jnp.int32),
            pltpu.VMEM((POOL_CH, 128), jnp.float32),
            pltpu.VMEM((2, 384), jnp.float32),
        ],
    )
    def k(h_hbm, b_hbm, out, bat_v, hrows, pool_v):
        c = lax.axis_index("c")
        s = lax.axis_index("s")
        w = c * 16 + s
        g0 = 2 * w

        pltpu.sync_copy(b_hbm, bat_v)

        def lower_bound(g):
            def bb(_, lohi):
                lo, hi = lohi
                mid = (lo + hi) // 2
                v = bat_v[pl.ds(mid * 16, 16)]
                big = v[0] >= g
                return (jnp.where(big, lo, mid + 1), jnp.where(big, mid, hi))
            bstar, _ = lax.fori_loop(
                0, 10, bb, (jnp.int32(0), jnp.int32(NPAD // 16)))
            bprev = jnp.maximum(bstar - 1, 0)
            v = bat_v[pl.ds(bprev * 16, 16)]
            cnt = jnp.int32(0)
            for l in range(16):
                cnt = cnt + jnp.where(v[l] < g, 1, 0).astype(jnp.int32)
            return bprev * 16 + cnt

        bounds = (lower_bound(g0), lower_bound(g0 + 1), lower_bound(g0 + 2))

        for gi in range(2):
            start = bounds[gi]
            end = bounds[gi + 1]
            cnt = end - start
            abase = (start // 8) * 8
            nch = (end - abase + (POOL_CH - 1)) // POOL_CH

            sums0 = tuple(jnp.zeros((16,), jnp.float32) for _ in range(8))
            maxs0 = tuple(jnp.zeros((16,), jnp.float32) for _ in range(8))

            def chunk(ci, accs):
                base = abase + ci * POOL_CH
                pltpu.sync_copy(h_hbm.at[pl.ds(base, POOL_CH)], hrows)
                lo = jnp.maximum(start - base, 0)
                hi = jnp.minimum(POOL_CH, end - base)

                def row(r, accs2):
                    sums, maxs = accs2
                    ns, nm = [], []
                    for l in range(8):
                        v = hrows[r, pl.ds(l * 16, 16)]
                        ns.append(sums[l] + v)
                        nm.append(jnp.maximum(maxs[l], v))
                    return (tuple(ns), tuple(nm))
                return lax.fori_loop(lo, hi, row, accs)
            sums, maxs = lax.fori_loop(0, nch, chunk, (sums0, maxs0))

            cnt_f = jnp.broadcast_to(cnt, (16,)).astype(jnp.float32)
            inv = 1.0 / jnp.maximum(cnt_f, 1.0)
            for l in range(8):
                pool_v[gi, pl.ds(l * 16, 16)] = sums[l] * inv
                pool_v[gi, pl.ds(128 + l * 16, 16)] = maxs[l]
                pool_v[gi, pl.ds(256 + l * 16, 16)] = sums[l]

        pltpu.sync_copy(pool_v, out.at[pl.ds(g0, 2)])

    return k(h, batch_pad)


_BM = 1024


def _encoder(x, W1, b1, W2, b2):
    def body(x_ref, w1_ref, b1_ref, w2_ref, b2_ref, o_ref):
        h = jnp.maximum(
            jnp.dot(x_ref[...], w1_ref[...],
                    preferred_element_type=jnp.float32) + b1_ref[...], 0.0)
        o_ref[...] = jnp.maximum(
            jnp.dot(h, w2_ref[...],
                    preferred_element_type=jnp.float32) + b2_ref[...], 0.0)

    return pl.pallas_call(
        body,
        grid=(NPAD // _BM,),
        in_specs=[
            pl.BlockSpec((_BM, 128), lambda i: (i, 0)),
            pl.BlockSpec((128, 128), lambda i: (0, 0)),
            pl.BlockSpec((1, 128), lambda i: (0, 0)),
            pl.BlockSpec((128, 128), lambda i: (0, 0)),
            pl.BlockSpec((1, 128), lambda i: (0, 0)),
        ],
        out_specs=pl.BlockSpec((_BM, 128), lambda i: (i, 0)),
        out_shape=jax.ShapeDtypeStruct((NPAD, 128), jnp.float32),
    )(x, W1, b1, W2, b2)


def _combine(p0, p1, h, Wrel, brel, Wroot):
    def body(p0_ref, p1_ref, h_ref, wr_ref, br_ref, wo_ref, o_ref):
        agg = p0_ref[...] + p1_ref[...]
        hv = h_ref[...]
        out = (jnp.dot(agg, wr_ref[...], preferred_element_type=jnp.float32)
               + br_ref[...]
               + jnp.dot(hv, wo_ref[...], preferred_element_type=jnp.float32))
        o_ref[...] = jnp.maximum(out, 0.0) + hv

    return pl.pallas_call(
        body,
        grid=(NPAD // _BM,),
        in_specs=[
            pl.BlockSpec((_BM, 128), lambda i: (i, 0)),
            pl.BlockSpec((_BM, 128), lambda i: (i, 0)),
            pl.BlockSpec((_BM, 128), lambda i: (i, 0)),
            pl.BlockSpec((128, 128), lambda i: (0, 0)),
            pl.BlockSpec((1, 128), lambda i: (0, 0)),
            pl.BlockSpec((128, 128), lambda i: (0, 0)),
        ],
        out_specs=pl.BlockSpec((_BM, 128), lambda i: (i, 0)),
        out_shape=jax.ShapeDtypeStruct((NPAD, 128), jnp.float32),
    )(p0, p1, h, Wrel, brel, Wroot)


def _head(pooled, gf, wg1, bg1, wg2, bg2, wp, wg, b1, w2, b2, w3, b3, w4, b4):
    def body(pool_ref, gf_ref, wg1_ref, bg1_ref, wg2_ref, bg2_ref, wp_ref,
             wgc_ref, b1_ref, w2_ref, b2_ref, w3_ref, b3_ref, w4_ref, b4_ref,
             o_ref):
        dot = functools.partial(jnp.dot, preferred_element_type=jnp.float32)
        g = jnp.maximum(dot(gf_ref[...], wg1_ref[...]) + bg1_ref[...], 0.0)
        g = jnp.maximum(dot(g, wg2_ref[...]) + bg2_ref[...], 0.0)
        z = jnp.maximum(
            dot(pool_ref[...], wp_ref[...]) + dot(g, wgc_ref[...])
            + b1_ref[...], 0.0)
        z = jnp.maximum(dot(z, w2_ref[...]) + b2_ref[...], 0.0)
        z = jnp.maximum(dot(z, w3_ref[...]) + b3_ref[...], 0.0)
        o_ref[...] = dot(z, w4_ref[...]) + b4_ref[...]

    return pl.pallas_call(
        body,
        out_shape=jax.ShapeDtypeStruct((64, 128), jnp.float32),
    )(pooled, gf, wg1, bg1, wg2, bg2, wp, wg, b1, w2, b2, w3, b3, w4, b4)


def kernel(x, edge_index, edge_attr, batch, global_features, params):
    del edge_attr

    xp = jnp.pad(x, ((0, NPAD - N), (0, 0)))
    pad_i = jnp.arange(EPAD - E, dtype=jnp.int32)
    src = jnp.concatenate(
        [edge_index[0], pad_i % N]).reshape(-1, ECHUNK)
    dst = jnp.concatenate(
        [edge_index[1], N + pad_i % (NPAD - N)]).reshape(-1, ECHUNK)
    batch_pad = jnp.concatenate(
        [batch, jnp.full((NPAD - N,), 64, jnp.int32)])

    row = lambda b: b.reshape(1, -1)

    W1, b1 = params['ne1']
    W2, b2 = params['ne2']
    h = _encoder(xp, W1, row(b1), W2, row(b2))

    for layer in params['convs']:
        Wrel, brel = layer['rel']
        p0, p1 = _edge_agg(h, src, dst)
        h = _combine(p0, p1, h, Wrel, row(brel), layer['root'])

    pooled = _pool(h, batch_pad)

    wg1, bg1 = params['ge1']
    wg2, bg2 = params['ge2']
    (w1c, b1c), (w2c, b2c), (w3c, b3c), (w4c, b4c) = params['cls']
    wp, wg = w1c[:384], w1c[384:]
    w4p = jnp.pad(w4c, ((0, 0), (0, 128 - w4c.shape[1])))
    b4p = jnp.pad(b4c, ((0, 128 - b4c.shape[0]),))

    out = _head(pooled, global_features, wg1, row(bg1), wg2, row(bg2),
                wp, wg, row(b1c), w2c, row(b2c), w3c, row(b3c), w4p, row(b4p))
    return out[:, :2]
